# Initial kernel scaffold; baseline (speedup 1.0000x reference)
import jax
import jax.numpy as jnp
from jax.experimental import pallas as pl


def kernel(x, w1, bn1_g, bn1_b, w2, b2, bn2_g, bn2_b, w3, b3):
    raise NotImplementedError("write your pallas kernel here")


# speedup vs baseline: 17.5332x; 17.5332x over previous
import functools

import jax
import jax.numpy as jnp
import numpy as np
from jax.experimental import pallas as pl
from jax.experimental.pallas import tpu as pltpu
from jax.experimental.pallas import tpu_sc as plsc

_B = 2
_N = 4096
_KP = 9
_NPT = _B * _N
_M = _NPT * _KP

_QT = 256
_NT = _N // _QT

_NC, _NS = 2, 16
_NW = _NC * _NS
_PER_W = _M // _NW
_CH = 128
_NCH = _PER_W // _CH


def _topk_body(xq_ref, xkt_ref, out_ref):
    b = pl.program_id(0)
    xq = xq_ref[0]
    xkt = xkt_ref[0]
    qsq = jnp.sum(xq * xq, axis=1, keepdims=True)
    ksq = jnp.sum(xkt * xkt, axis=0, keepdims=True)
    prod = jax.lax.dot_general(
        xq.astype(jnp.bfloat16), xkt.astype(jnp.bfloat16),
        (((1,), (0,)), ((), ())),
        preferred_element_type=jnp.float32)
    dist = (-qsq - ksq) + 2.0 * prod
    col = jax.lax.broadcasted_iota(jnp.int32, (_QT, _N), 1)
    neg = jnp.float32(-3.0e38)
    outs = []
    for j in range(_KP + 1):
        m = jnp.max(dist, axis=1, keepdims=True)
        cand = jnp.where(dist == m, col, _N)
        amin = jnp.min(cand, axis=1, keepdims=True)
        if j > 0:
            outs.append(amin)
        dist = jnp.where(col == amin, neg, dist)
    idx = jnp.concatenate(outs, axis=1) + b * _N
    out_ref[0] = idx


def _sc_gather_body(table_hbm, idx_hbm, out_hbm, idx_v, rows_v, sem):
    wid = jax.lax.axis_index("s") * _NC + jax.lax.axis_index("c")
    base = wid * _PER_W
    pltpu.sync_copy(idx_hbm.at[wid], idx_v)
    copies = [
        pltpu.async_copy(table_hbm.at[idx_v.at[j]],
                         rows_v.at[pl.ds(j * _CH, _CH)], sem)
        for j in range(_NCH)
    ]
    for c in copies:
        c.wait()
    pltpu.sync_copy(rows_v, out_hbm.at[pl.ds(base, _PER_W)])


def _mlp_body(xnt_ref, xt_ref, w1_ref, g1_ref, be1_ref, w2_ref, b2_ref,
              g2_ref, be2_ref, w3_ref, b3_ref, out_ref):
    cxb = xt_ref[0:1, :]
    cyb = xt_ref[1:2, :]
    czb = xt_ref[2:3, :]
    rx = xnt_ref[0] - cxb
    ry = xnt_ref[1] - cyb
    rz = xnt_ref[2] - czb

    phi = jnp.arctan2(ry, rx) / (2.0 * np.pi) + 0.5

    riot = jax.lax.broadcasted_iota(jnp.int32, (_KP, _NPT), 0)
    pw = phi
    sx, sy, sz = [], [], []
    big = jnp.float32(1e9)
    for _ in range(_KP):
        m = jnp.min(pw, axis=0, keepdims=True)
        eq = pw == m
        cand = jnp.where(eq, riot, _KP)
        amin = jnp.min(cand, axis=0, keepdims=True)
        oh = riot == amin
        ohf = oh.astype(jnp.float32)
        sx.append(jnp.sum(ohf * rx, axis=0, keepdims=True))
        sy.append(jnp.sum(ohf * ry, axis=0, keepdims=True))
        sz.append(jnp.sum(ohf * rz, axis=0, keepdims=True))
        pw = jnp.where(oh, big, pw)
    px = jnp.concatenate(sx, axis=0)
    py = jnp.concatenate(sy, axis=0)
    pz = jnp.concatenate(sz, axis=0)
    qx = jnp.concatenate([px[1:], px[:1]], axis=0)
    qy = jnp.concatenate([py[1:], py[:1]], axis=0)
    qz = jnp.concatenate([pz[1:], pz[:1]], axis=0)

    cxx = (px + qx) * 0.5
    cyy = (py + qy) * 0.5
    czz = (pz + qz) * 0.5
    nx = py * qz - pz * qy + 1e-5
    ny = pz * qx - px * qz + 1e-5
    nz = px * qy - py * qx + 1e-5
    nrm = jnp.sqrt(nx * nx + ny * ny + nz * nz)
    nx = nx / nrm
    ny = ny / nrm
    nz = nz / nrm
    s = jnp.where(nx[0:1, :] > 0, jnp.float32(1.0), jnp.float32(-1.0))
    nx = nx * s
    ny = ny * s
    nz = nz * s
    pos = (cxx * nx + cyy * ny + czz * nz) / np.sqrt(3.0)

    feats = [cxx, cyy, czz, nx, ny, nz, pos]

    def dense(fin, w_ref, b_ref=None):
        out = []
        for o in range(7):
            acc = fin[0] * w_ref[0, o * 7 + 0]
            for c in range(1, 7):
                acc = acc + fin[c] * w_ref[0, o * 7 + c]
            if b_ref is not None:
                acc = acc + b_ref[0, o]
            out.append(acc)
        return out

    def bn_relu(fin, g_ref, b_ref):
        out = []
        for o in range(7):
            f = fin[o]
            m = jnp.mean(f)
            v = jnp.mean((f - m) ** 2)
            h = g_ref[0, o] * (f - m) / jnp.sqrt(v + 1e-5) + b_ref[0, o]
            out.append(jnp.maximum(h, 0.0))
        return out

    f1 = bn_relu(dense(feats, w1_ref), g1_ref, be1_ref)
    f2 = bn_relu(dense(f1, w2_ref, b2_ref), g2_ref, be2_ref)
    f3 = dense(f2, w3_ref, b3_ref)
    rows = [jnp.max(f3[o], axis=0, keepdims=True) for o in range(7)]
    out_ref[...] = jnp.concatenate(rows, axis=0)


def _sc_gather_call(table, idx3):
    call = pl.kernel(
        _sc_gather_body,
        out_type=jax.ShapeDtypeStruct((_M, 16), jnp.float32),
        mesh=plsc.VectorSubcoreMesh(core_axis_name="c", subcore_axis_name="s",
                                    num_cores=_NC, num_subcores=_NS),
        scratch_types=[
            pltpu.VMEM((_NCH, _CH), jnp.int32),
            pltpu.VMEM((_PER_W, 16), jnp.float32),
            pltpu.SemaphoreType.DMA,
        ],
        compiler_params=pltpu.CompilerParams(use_tc_tiling_on_sc=False),
    )
    return call(table, idx3)


def kernel(x, w1, bn1_g, bn1_b, w2, b2, bn2_g, bn2_b, w3, b3):
    f32 = jnp.float32
    x_pad = jnp.zeros((_B, _N, 8), f32).at[..., :3].set(x)
    x_t = jnp.transpose(x_pad, (0, 2, 1))

    idx = pl.pallas_call(
        _topk_body,
        grid=(_B, _NT),
        in_specs=[
            pl.BlockSpec((1, _QT, 8), lambda b, t: (b, t, 0)),
            pl.BlockSpec((1, 8, _N), lambda b, t: (b, 0, 0)),
        ],
        out_specs=pl.BlockSpec((1, _QT, _KP), lambda b, t: (b, t, 0)),
        out_shape=jax.ShapeDtypeStruct((_B, _N, _KP), jnp.int32),
    )(x_pad, x_t)

    table = jnp.zeros((_NPT, 16), f32).at[:, :3].set(x.reshape(_NPT, 3))
    idx3 = idx.reshape(_NW, _NCH, _CH)
    neigh = _sc_gather_call(table, idx3)

    xnt = neigh[:, :3].reshape(_NPT, _KP, 3).transpose(2, 1, 0)
    xt = x.reshape(_NPT, 3).T

    smem = pl.BlockSpec(memory_space=pltpu.SMEM)
    vmem = pl.BlockSpec(memory_space=pltpu.VMEM)
    fout = pl.pallas_call(
        _mlp_body,
        in_specs=[vmem, vmem] + [smem] * 9,
        out_specs=vmem,
        out_shape=jax.ShapeDtypeStruct((7, _NPT), f32),
    )(xnt, xt,
      w1.reshape(1, 49), bn1_g.reshape(1, 7), bn1_b.reshape(1, 7),
      w2.reshape(1, 49), b2.reshape(1, 7), bn2_g.reshape(1, 7),
      bn2_b.reshape(1, 7), w3.reshape(1, 49), b3.reshape(1, 7))

    f = fout.T.reshape(_B, _N, 7)
    return jnp.concatenate([x, f], axis=-1)
